# baseline (speedup 1.0000x reference)
import functools

import jax
import jax.numpy as jnp
from jax import lax
from jax.experimental import pallas as pl
from jax.experimental.pallas import tpu as pltpu
from jax.experimental.pallas import tpu_sc as plsc

N = 10000
D = 128
L = 16
NC = 2
NS = 16
NW = NC * NS
CHUNK = 128
ROWS_MAIN = (N // (NS * 8)) * 8
TAIL = N - NS * ROWS_MAIN
TAIL_OFF = NS * ROWS_MAIN
_SLAB = [(o, min(CHUNK, ROWS_MAIN - o)) for o in range(0, ROWS_MAIN, CHUNK)]

_mesh = plsc.VectorSubcoreMesh(core_axis_name="c", subcore_axis_name="s")


def _deg_body(nch0, nch1, dst0_hbm, w0_hbm, dst1_hbm, w1_hbm, deg_out,
              dst_all, w_all, zb, deg_sh):
    c = lax.axis_index("c")
    s = lax.axis_index("s")
    r0 = s * ROWS_MAIN

    def zb_body(i, _):
        zb[pl.ds(i * L, L)] = jnp.zeros((L,), jnp.float32)
        return 0

    lax.fori_loop(0, ROWS_MAIN // L, zb_body, 0)
    pltpu.sync_copy(zb, deg_sh.at[pl.ds(r0, ROWS_MAIN)])

    @pl.when(s == 0)
    def _():
        pltpu.sync_copy(zb.at[pl.ds(0, TAIL)], deg_sh.at[pl.ds(TAIL_OFF, TAIL)])

    plsc.subcore_barrier()

    def run(dst3, w3, nch):
        pltpu.sync_copy(dst3.at[s], dst_all.at[pl.ds(0, nch)])
        pltpu.sync_copy(w3.at[s], w_all.at[pl.ds(0, nch)])

        def chunk_body(k, _):
            pltpu.sync_copy(w_all.at[k], deg_sh.at[dst_all.at[k]], add=True)
            return 0

        lax.fori_loop(0, nch, chunk_body, 0)

    @pl.when(c == 0)
    def _():
        run(dst0_hbm, w0_hbm, nch0)

    @pl.when(c == 1)
    def _():
        run(dst1_hbm, w1_hbm, nch1)

    plsc.subcore_barrier()

    pltpu.sync_copy(deg_sh.at[pl.ds(r0, ROWS_MAIN)], zb)
    pltpu.sync_copy(zb, deg_out.at[pl.ds(c * N + r0, ROWS_MAIN)])

    @pl.when(s == 0)
    def _():
        pltpu.sync_copy(deg_sh.at[pl.ds(TAIL_OFF, TAIL)], zb.at[pl.ds(0, TAIL)])
        pltpu.sync_copy(zb.at[pl.ds(0, TAIL)], deg_out.at[pl.ds(c * N + TAIL_OFF, TAIL)])


def _deg_call(dst0, w0, dst1, w1, nch0, nch1):
    return pl.kernel(
        functools.partial(_deg_body, nch0, nch1),
        out_type=jax.ShapeDtypeStruct((NC * N,), jnp.float32),
        mesh=_mesh,
        scratch_types=[
            pltpu.VMEM((max(nch0, nch1), CHUNK), jnp.int32),
            pltpu.VMEM((max(nch0, nch1), CHUNK), jnp.float32),
            pltpu.VMEM((ROWS_MAIN,), jnp.float32),
            pltpu.VMEM_SHARED((N,), jnp.float32),
        ],
    )(dst0, w0, dst1, w1)


def _prop_body(nch0, nch1, hp_hbm, src0_hbm, dst0_hbm, w0_hbm,
               src1_hbm, dst1_hbm, w1_hbm, part_out,
               dst_all, idx_a, idx_b, w_a, w_b, rows_a, rows_b,
               acc_sh, sem_ga, sem_gb, sem_sa, sem_sb, sem_pa, sem_pb):
    c = lax.axis_index("c")
    s = lax.axis_index("s")
    r0 = s * ROWS_MAIN

    def zero_rows(e, _):
        z = jnp.zeros((L,), jnp.float32)
        for j in range(D // L):
            rows_a[e, pl.ds(j * L, L)] = z
        return 0

    lax.fori_loop(0, CHUNK, zero_rows, 0)
    for o, nr in _SLAB:
        pltpu.sync_copy(rows_a.at[pl.ds(0, nr)], acc_sh.at[pl.ds(r0 + o, nr)])

    @pl.when(s == 0)
    def _():
        pltpu.sync_copy(rows_a.at[pl.ds(0, TAIL)], acc_sh.at[pl.ds(TAIL_OFF, TAIL)])

    plsc.subcore_barrier()

    GSPLIT = 4
    GH = CHUNK // GSPLIT

    def gather(idx_r, rows_r, sem):
        for g in range(GSPLIT):
            pltpu.async_copy(hp_hbm.at[idx_r.at[pl.ds(g * GH, GH)]],
                             rows_r.at[pl.ds(g * GH, GH)], sem)

    def gather_wait(idx_r, rows_r, sem):
        for g in range(GSPLIT):
            pltpu.make_async_copy(hp_hbm.at[idx_r.at[pl.ds(g * GH, GH)]],
                                  rows_r.at[pl.ds(g * GH, GH)], sem).wait()

    def scatter(k, rows_r, sem):
        pltpu.async_copy(rows_r, acc_sh.at[dst_all.at[k]], sem, add=True)

    def scatter_wait(k, rows_r, sem):
        pltpu.make_async_copy(rows_r, acc_sh.at[dst_all.at[k]], sem).wait()

    def mul(w_r, rows_r):
        def edge_body(e, _):
            wsc = w_r[pl.ds(e, L)][0]
            for j in range(D // L):
                rows_r[e, pl.ds(j * L, L)] = rows_r[e, pl.ds(j * L, L)] * wsc
            return 0

        lax.fori_loop(0, CHUNK, edge_body, 0, unroll=4)

    def run(src3, dst3, w3, nch):
        last = nch - 1
        pltpu.sync_copy(dst3.at[s], dst_all.at[pl.ds(0, nch)])

        def prefetch(k, idx_r, w_r, sem):
            pltpu.async_copy(src3.at[s, k], idx_r, sem)
            pltpu.async_copy(w3.at[s, k], w_r.at[pl.ds(0, CHUNK)], sem)

        def prefetch_wait(k, idx_r, w_r, sem):
            pltpu.make_async_copy(src3.at[s, k], idx_r, sem).wait()
            pltpu.make_async_copy(w3.at[s, k], w_r.at[pl.ds(0, CHUNK)], sem).wait()

        prefetch(0, idx_a, w_a, sem_pa)
        prefetch_wait(0, idx_a, w_a, sem_pa)
        gather(idx_a, rows_a, sem_ga)
        prefetch(1, idx_b, w_b, sem_pb)

        def pair_body(i, _):
            k = 2 * i

            @pl.when(i > 0)
            def _():
                scatter_wait(k - 1, rows_b, sem_sb)

            prefetch_wait(k + 1, idx_b, w_b, sem_pb)
            gather(idx_b, rows_b, sem_gb)
            gather_wait(idx_a, rows_a, sem_ga)
            mul(w_a, rows_a)
            scatter(k, rows_a, sem_sa)
            prefetch(jnp.minimum(k + 2, last), idx_a, w_a, sem_pa)
            gather_wait(idx_b, rows_b, sem_gb)
            scatter_wait(k, rows_a, sem_sa)
            prefetch_wait(jnp.minimum(k + 2, last), idx_a, w_a, sem_pa)
            gather(idx_a, rows_a, sem_ga)
            mul(w_b, rows_b)
            scatter(k + 1, rows_b, sem_sb)
            prefetch(jnp.minimum(k + 3, last), idx_b, w_b, sem_pb)
            return 0

        lax.fori_loop(0, nch // 2, pair_body, 0)
        scatter_wait(last, rows_b, sem_sb)
        prefetch_wait(last, idx_b, w_b, sem_pb)
        gather_wait(idx_a, rows_a, sem_ga)

    @pl.when(c == 0)
    def _():
        run(src0_hbm, dst0_hbm, w0_hbm, nch0)

    @pl.when(c == 1)
    def _():
        run(src1_hbm, dst1_hbm, w1_hbm, nch1)

    plsc.subcore_barrier()

    for o, nr in _SLAB:
        pltpu.sync_copy(acc_sh.at[pl.ds(r0 + o, nr)], rows_a.at[pl.ds(0, nr)])
        pltpu.sync_copy(rows_a.at[pl.ds(0, nr)], part_out.at[c, pl.ds(r0 + o, nr)])

    @pl.when(s == 0)
    def _():
        pltpu.sync_copy(acc_sh.at[pl.ds(TAIL_OFF, TAIL)], rows_a.at[pl.ds(0, TAIL)])
        pltpu.sync_copy(rows_a.at[pl.ds(0, TAIL)], part_out.at[c, pl.ds(TAIL_OFF, TAIL)])


def _prop_call(hp, edges, nch0, nch1):
    src0, dst0, w0, src1, dst1, w1 = edges
    return pl.kernel(
        functools.partial(_prop_body, nch0, nch1),
        out_type=jax.ShapeDtypeStruct((NC, N, D), jnp.float32),
        mesh=_mesh,
        scratch_types=[
            pltpu.VMEM((max(nch0, nch1), CHUNK), jnp.int32),
            pltpu.VMEM((CHUNK,), jnp.int32),
            pltpu.VMEM((CHUNK,), jnp.int32),
            pltpu.VMEM((CHUNK + L,), jnp.float32),
            pltpu.VMEM((CHUNK + L,), jnp.float32),
            pltpu.VMEM((CHUNK, D), jnp.float32),
            pltpu.VMEM((CHUNK, D), jnp.float32),
            pltpu.VMEM_SHARED((N, D), jnp.float32),
            pltpu.SemaphoreType.DMA,
            pltpu.SemaphoreType.DMA,
            pltpu.SemaphoreType.DMA,
            pltpu.SemaphoreType.DMA,
            pltpu.SemaphoreType.DMA,
            pltpu.SemaphoreType.DMA,
        ],
    )(hp, src0, dst0, w0, src1, dst1, w1)


_BLK = 1000


def _dinv_block(deg_ref):
    deg = deg_ref[0, :, 0] + deg_ref[1, :, 0] + 1.0
    return jnp.where(deg > 0, lax.rsqrt(deg), 0.0)


def _mm_scale_body(deg_ref, x_ref, w_ref, o_ref):
    dinv = _dinv_block(deg_ref)
    h = jnp.dot(x_ref[...], w_ref[...], preferred_element_type=jnp.float32)
    o_ref[...] = h * dinv[:, None]


def _mm_scale(deg3, x, W1):
    return pl.pallas_call(
        _mm_scale_body,
        grid=(N // _BLK,),
        in_specs=[
            pl.BlockSpec((NC, _BLK, 1), lambda i: (0, i, 0)),
            pl.BlockSpec((_BLK, D), lambda i: (i, 0)),
            pl.BlockSpec((D, D), lambda i: (0, 0)),
        ],
        out_specs=pl.BlockSpec((_BLK, D), lambda i: (i, 0)),
        out_shape=jax.ShapeDtypeStruct((N, D), jnp.float32),
    )(deg3, x, W1)


def _mid_body(deg_ref, s_ref, hp_ref, b_ref, w_ref, o_ref):
    dinv = _dinv_block(deg_ref)
    pre = (s_ref[0] + s_ref[1] + hp_ref[...]) * dinv[:, None] + b_ref[...]
    h1 = jnp.maximum(pre, 0.0)
    g = jnp.dot(h1, w_ref[...], preferred_element_type=jnp.float32)
    o_ref[...] = g * dinv[:, None]


def _mid_layer(deg3, S1, h1p, b1, W2):
    return pl.pallas_call(
        _mid_body,
        grid=(N // _BLK,),
        in_specs=[
            pl.BlockSpec((NC, _BLK, 1), lambda i: (0, i, 0)),
            pl.BlockSpec((NC, _BLK, D), lambda i: (0, i, 0)),
            pl.BlockSpec((_BLK, D), lambda i: (i, 0)),
            pl.BlockSpec((1, D), lambda i: (0, 0)),
            pl.BlockSpec((D, D), lambda i: (0, 0)),
        ],
        out_specs=pl.BlockSpec((_BLK, D), lambda i: (i, 0)),
        out_shape=jax.ShapeDtypeStruct((N, D), jnp.float32),
    )(deg3, S1, h1p, b1.reshape(1, D), W2)


def _final_body(deg_ref, s_ref, hp_ref, b_ref, o_ref):
    dinv = _dinv_block(deg_ref)
    o_ref[...] = (s_ref[0] + s_ref[1] + hp_ref[...]) * dinv[:, None] + b_ref[...]


def _final_layer(deg3, S2, h2p, b2):
    return pl.pallas_call(
        _final_body,
        grid=(N // _BLK,),
        in_specs=[
            pl.BlockSpec((NC, _BLK, 1), lambda i: (0, i, 0)),
            pl.BlockSpec((NC, _BLK, D), lambda i: (0, i, 0)),
            pl.BlockSpec((_BLK, D), lambda i: (i, 0)),
            pl.BlockSpec((1, D), lambda i: (0, 0)),
        ],
        out_specs=pl.BlockSpec((_BLK, D), lambda i: (i, 0)),
        out_shape=jax.ShapeDtypeStruct((N, D), jnp.float32),
    )(deg3, S2, h2p, b2.reshape(1, D))


_CORE0_FRAC = 0.76


def kernel(x, attn_edge_index, attn_edge_weight, W1, b1, W2, b2):
    E = attn_edge_index.shape[1]
    t = -(-E // (NS * CHUNK))
    nch0 = max(2, int(round(t * _CORE0_FRAC / 2)) * 2)
    nch1 = max(2, ((t - nch0 + 1) // 2) * 2)
    e0 = NS * nch0 * CHUNK
    e_pad = e0 + NS * nch1 * CHUNK
    pad = e_pad - E

    src_p = jnp.pad(attn_edge_index[0], (0, pad))
    dst_p = jnp.pad(attn_edge_index[1], (0, pad))
    w_p = jnp.pad(attn_edge_weight, (0, pad))
    edges = (
        src_p[:e0].reshape(NS, nch0, CHUNK),
        dst_p[:e0].reshape(NS, nch0, CHUNK),
        w_p[:e0].reshape(NS, nch0, CHUNK),
        src_p[e0:].reshape(NS, nch1, CHUNK),
        dst_p[e0:].reshape(NS, nch1, CHUNK),
        w_p[e0:].reshape(NS, nch1, CHUNK),
    )

    degflat = _deg_call(edges[1], edges[2], edges[4], edges[5], nch0, nch1)
    deg3 = degflat.reshape(NC, N, 1)
    h1p = _mm_scale(deg3, x, W1)
    S1 = _prop_call(h1p, edges, nch0, nch1)
    h2p = _mid_layer(deg3, S1, h1p, b1, W2)
    S2 = _prop_call(h2p, edges, nch0, nch1)
    return _final_layer(deg3, S2, h2p, b2)


# speedup vs baseline: 1.0212x; 1.0212x over previous
import functools

import jax
import jax.numpy as jnp
from jax import lax
from jax.experimental import pallas as pl
from jax.experimental.pallas import tpu as pltpu
from jax.experimental.pallas import tpu_sc as plsc

N = 10000
D = 128
L = 16
NC = 2
NS = 16
NW = NC * NS
CHUNK = 128
ROWS_MAIN = (N // (NS * 8)) * 8
TAIL = N - NS * ROWS_MAIN
TAIL_OFF = NS * ROWS_MAIN
_SLAB = [(o, min(CHUNK, ROWS_MAIN - o)) for o in range(0, ROWS_MAIN, CHUNK)]

_mesh = plsc.VectorSubcoreMesh(core_axis_name="c", subcore_axis_name="s")


def _deg_body(nch0, nch1, dst0_hbm, w0_hbm, dst1_hbm, w1_hbm, deg_out,
              dst_all, w_all, zb, deg_sh):
    c = lax.axis_index("c")
    s = lax.axis_index("s")
    r0 = s * ROWS_MAIN

    def zb_body(i, _):
        zb[pl.ds(i * L, L)] = jnp.zeros((L,), jnp.float32)
        return 0

    lax.fori_loop(0, ROWS_MAIN // L, zb_body, 0)
    pltpu.sync_copy(zb, deg_sh.at[pl.ds(r0, ROWS_MAIN)])

    @pl.when(s == 0)
    def _():
        pltpu.sync_copy(zb.at[pl.ds(0, TAIL)], deg_sh.at[pl.ds(TAIL_OFF, TAIL)])

    plsc.subcore_barrier()

    def run(dst3, w3, nch):
        pltpu.sync_copy(dst3.at[s], dst_all.at[pl.ds(0, nch)])
        pltpu.sync_copy(w3.at[s], w_all.at[pl.ds(0, nch)])

        def chunk_body(k, _):
            pltpu.sync_copy(w_all.at[k], deg_sh.at[dst_all.at[k]], add=True)
            return 0

        lax.fori_loop(0, nch, chunk_body, 0)

    @pl.when(c == 0)
    def _():
        run(dst0_hbm, w0_hbm, nch0)

    @pl.when(c == 1)
    def _():
        run(dst1_hbm, w1_hbm, nch1)

    plsc.subcore_barrier()

    pltpu.sync_copy(deg_sh.at[pl.ds(r0, ROWS_MAIN)], zb)
    pltpu.sync_copy(zb, deg_out.at[pl.ds(c * N + r0, ROWS_MAIN)])

    @pl.when(s == 0)
    def _():
        pltpu.sync_copy(deg_sh.at[pl.ds(TAIL_OFF, TAIL)], zb.at[pl.ds(0, TAIL)])
        pltpu.sync_copy(zb.at[pl.ds(0, TAIL)], deg_out.at[pl.ds(c * N + TAIL_OFF, TAIL)])


def _deg_call(dst0, w0, dst1, w1, nch0, nch1):
    return pl.kernel(
        functools.partial(_deg_body, nch0, nch1),
        out_type=jax.ShapeDtypeStruct((NC * N,), jnp.float32),
        mesh=_mesh,
        scratch_types=[
            pltpu.VMEM((max(nch0, nch1), CHUNK), jnp.int32),
            pltpu.VMEM((max(nch0, nch1), CHUNK), jnp.float32),
            pltpu.VMEM((ROWS_MAIN,), jnp.float32),
            pltpu.VMEM_SHARED((N,), jnp.float32),
        ],
    )(dst0, w0, dst1, w1)


def _prop_body(nch0, nch1, hp_hbm, src0_hbm, dst0_hbm, w0_hbm,
               src1_hbm, dst1_hbm, w1_hbm, part_out,
               dst_all, idx_a, idx_b, w_a, w_b, rows_a, rows_b,
               acc_sh, sem_ga, sem_gb, sem_sa, sem_sb, sem_pa, sem_pb):
    c = lax.axis_index("c")
    s = lax.axis_index("s")
    r0 = s * ROWS_MAIN

    def zero_rows(e, _):
        z = jnp.zeros((L,), jnp.float32)
        for j in range(D // L):
            rows_a[e, pl.ds(j * L, L)] = z
        return 0

    lax.fori_loop(0, CHUNK, zero_rows, 0)
    for o, nr in _SLAB:
        pltpu.sync_copy(rows_a.at[pl.ds(0, nr)], acc_sh.at[pl.ds(r0 + o, nr)])

    @pl.when(s == 0)
    def _():
        pltpu.sync_copy(rows_a.at[pl.ds(0, TAIL)], acc_sh.at[pl.ds(TAIL_OFF, TAIL)])

    plsc.subcore_barrier()

    def make_gather(nsplit):
        gh = CHUNK // nsplit

        def gather(idx_r, rows_r, sem):
            for g in range(nsplit):
                pltpu.async_copy(hp_hbm.at[idx_r.at[pl.ds(g * gh, gh)]],
                                 rows_r.at[pl.ds(g * gh, gh)], sem)

        def gather_wait(idx_r, rows_r, sem):
            for g in range(nsplit):
                pltpu.make_async_copy(hp_hbm.at[idx_r.at[pl.ds(g * gh, gh)]],
                                      rows_r.at[pl.ds(g * gh, gh)], sem).wait()

        return gather, gather_wait

    def scatter(k, rows_r, sem):
        pltpu.async_copy(rows_r, acc_sh.at[dst_all.at[k]], sem, add=True)

    def scatter_wait(k, rows_r, sem):
        pltpu.make_async_copy(rows_r, acc_sh.at[dst_all.at[k]], sem).wait()

    def mul(w_r, rows_r):
        def edge_body(e, _):
            wsc = w_r[pl.ds(e, L)][0]
            for j in range(D // L):
                rows_r[e, pl.ds(j * L, L)] = rows_r[e, pl.ds(j * L, L)] * wsc
            return 0

        lax.fori_loop(0, CHUNK, edge_body, 0, unroll=4)

    def run(src3, dst3, w3, nch, nsplit):
        gather, gather_wait = make_gather(nsplit)
        last = nch - 1
        pltpu.sync_copy(dst3.at[s], dst_all.at[pl.ds(0, nch)])

        def prefetch(k, idx_r, w_r, sem):
            pltpu.async_copy(src3.at[s, k], idx_r, sem)
            pltpu.async_copy(w3.at[s, k], w_r.at[pl.ds(0, CHUNK)], sem)

        def prefetch_wait(k, idx_r, w_r, sem):
            pltpu.make_async_copy(src3.at[s, k], idx_r, sem).wait()
            pltpu.make_async_copy(w3.at[s, k], w_r.at[pl.ds(0, CHUNK)], sem).wait()

        prefetch(0, idx_a, w_a, sem_pa)
        prefetch_wait(0, idx_a, w_a, sem_pa)
        gather(idx_a, rows_a, sem_ga)
        prefetch(1, idx_b, w_b, sem_pb)

        def pair_body(i, _):
            k = 2 * i

            @pl.when(i > 0)
            def _():
                scatter_wait(k - 1, rows_b, sem_sb)

            prefetch_wait(k + 1, idx_b, w_b, sem_pb)
            gather(idx_b, rows_b, sem_gb)
            gather_wait(idx_a, rows_a, sem_ga)
            mul(w_a, rows_a)
            scatter(k, rows_a, sem_sa)
            prefetch(jnp.minimum(k + 2, last), idx_a, w_a, sem_pa)
            gather_wait(idx_b, rows_b, sem_gb)
            scatter_wait(k, rows_a, sem_sa)
            prefetch_wait(jnp.minimum(k + 2, last), idx_a, w_a, sem_pa)
            gather(idx_a, rows_a, sem_ga)
            mul(w_b, rows_b)
            scatter(k + 1, rows_b, sem_sb)
            prefetch(jnp.minimum(k + 3, last), idx_b, w_b, sem_pb)
            return 0

        lax.fori_loop(0, nch // 2, pair_body, 0)
        scatter_wait(last, rows_b, sem_sb)
        prefetch_wait(last, idx_b, w_b, sem_pb)
        gather_wait(idx_a, rows_a, sem_ga)

    @pl.when(c == 0)
    def _():
        run(src0_hbm, dst0_hbm, w0_hbm, nch0, 1)

    @pl.when(c == 1)
    def _():
        run(src1_hbm, dst1_hbm, w1_hbm, nch1, 2)

    plsc.subcore_barrier()

    for o, nr in _SLAB:
        pltpu.sync_copy(acc_sh.at[pl.ds(r0 + o, nr)], rows_a.at[pl.ds(0, nr)])
        pltpu.sync_copy(rows_a.at[pl.ds(0, nr)], part_out.at[c, pl.ds(r0 + o, nr)])

    @pl.when(s == 0)
    def _():
        pltpu.sync_copy(acc_sh.at[pl.ds(TAIL_OFF, TAIL)], rows_a.at[pl.ds(0, TAIL)])
        pltpu.sync_copy(rows_a.at[pl.ds(0, TAIL)], part_out.at[c, pl.ds(TAIL_OFF, TAIL)])


def _prop_call(hp, edges, nch0, nch1):
    src0, dst0, w0, src1, dst1, w1 = edges
    return pl.kernel(
        functools.partial(_prop_body, nch0, nch1),
        out_type=jax.ShapeDtypeStruct((NC, N, D), jnp.float32),
        mesh=_mesh,
        scratch_types=[
            pltpu.VMEM((max(nch0, nch1), CHUNK), jnp.int32),
            pltpu.VMEM((CHUNK,), jnp.int32),
            pltpu.VMEM((CHUNK,), jnp.int32),
            pltpu.VMEM((CHUNK + L,), jnp.float32),
            pltpu.VMEM((CHUNK + L,), jnp.float32),
            pltpu.VMEM((CHUNK, D), jnp.float32),
            pltpu.VMEM((CHUNK, D), jnp.float32),
            pltpu.VMEM_SHARED((N, D), jnp.float32),
            pltpu.SemaphoreType.DMA,
            pltpu.SemaphoreType.DMA,
            pltpu.SemaphoreType.DMA,
            pltpu.SemaphoreType.DMA,
            pltpu.SemaphoreType.DMA,
            pltpu.SemaphoreType.DMA,
        ],
    )(hp, src0, dst0, w0, src1, dst1, w1)


_BLK = 1000


def _dinv_block(deg_ref):
    deg = deg_ref[0, :, 0] + deg_ref[1, :, 0] + 1.0
    return jnp.where(deg > 0, lax.rsqrt(deg), 0.0)


def _mm_scale_body(deg_ref, x_ref, w_ref, o_ref):
    dinv = _dinv_block(deg_ref)
    h = jnp.dot(x_ref[...], w_ref[...], preferred_element_type=jnp.float32)
    o_ref[...] = h * dinv[:, None]


def _mm_scale(deg3, x, W1):
    return pl.pallas_call(
        _mm_scale_body,
        grid=(N // _BLK,),
        in_specs=[
            pl.BlockSpec((NC, _BLK, 1), lambda i: (0, i, 0)),
            pl.BlockSpec((_BLK, D), lambda i: (i, 0)),
            pl.BlockSpec((D, D), lambda i: (0, 0)),
        ],
        out_specs=pl.BlockSpec((_BLK, D), lambda i: (i, 0)),
        out_shape=jax.ShapeDtypeStruct((N, D), jnp.float32),
    )(deg3, x, W1)


def _mid_body(deg_ref, s_ref, hp_ref, b_ref, w_ref, o_ref):
    dinv = _dinv_block(deg_ref)
    pre = (s_ref[0] + s_ref[1] + hp_ref[...]) * dinv[:, None] + b_ref[...]
    h1 = jnp.maximum(pre, 0.0)
    g = jnp.dot(h1, w_ref[...], preferred_element_type=jnp.float32)
    o_ref[...] = g * dinv[:, None]


def _mid_layer(deg3, S1, h1p, b1, W2):
    return pl.pallas_call(
        _mid_body,
        grid=(N // _BLK,),
        in_specs=[
            pl.BlockSpec((NC, _BLK, 1), lambda i: (0, i, 0)),
            pl.BlockSpec((NC, _BLK, D), lambda i: (0, i, 0)),
            pl.BlockSpec((_BLK, D), lambda i: (i, 0)),
            pl.BlockSpec((1, D), lambda i: (0, 0)),
            pl.BlockSpec((D, D), lambda i: (0, 0)),
        ],
        out_specs=pl.BlockSpec((_BLK, D), lambda i: (i, 0)),
        out_shape=jax.ShapeDtypeStruct((N, D), jnp.float32),
    )(deg3, S1, h1p, b1.reshape(1, D), W2)


def _final_body(deg_ref, s_ref, hp_ref, b_ref, o_ref):
    dinv = _dinv_block(deg_ref)
    o_ref[...] = (s_ref[0] + s_ref[1] + hp_ref[...]) * dinv[:, None] + b_ref[...]


def _final_layer(deg3, S2, h2p, b2):
    return pl.pallas_call(
        _final_body,
        grid=(N // _BLK,),
        in_specs=[
            pl.BlockSpec((NC, _BLK, 1), lambda i: (0, i, 0)),
            pl.BlockSpec((NC, _BLK, D), lambda i: (0, i, 0)),
            pl.BlockSpec((_BLK, D), lambda i: (i, 0)),
            pl.BlockSpec((1, D), lambda i: (0, 0)),
        ],
        out_specs=pl.BlockSpec((_BLK, D), lambda i: (i, 0)),
        out_shape=jax.ShapeDtypeStruct((N, D), jnp.float32),
    )(deg3, S2, h2p, b2.reshape(1, D))


_CORE0_FRAC = 0.76


def kernel(x, attn_edge_index, attn_edge_weight, W1, b1, W2, b2):
    E = attn_edge_index.shape[1]
    t = -(-E // (NS * CHUNK))
    nch0 = max(2, int(round(t * _CORE0_FRAC / 2)) * 2)
    nch1 = max(2, ((t - nch0 + 1) // 2) * 2)
    e0 = NS * nch0 * CHUNK
    e_pad = e0 + NS * nch1 * CHUNK
    pad = e_pad - E

    src_p = jnp.pad(attn_edge_index[0], (0, pad))
    dst_p = jnp.pad(attn_edge_index[1], (0, pad))
    w_p = jnp.pad(attn_edge_weight, (0, pad))
    edges = (
        src_p[:e0].reshape(NS, nch0, CHUNK),
        dst_p[:e0].reshape(NS, nch0, CHUNK),
        w_p[:e0].reshape(NS, nch0, CHUNK),
        src_p[e0:].reshape(NS, nch1, CHUNK),
        dst_p[e0:].reshape(NS, nch1, CHUNK),
        w_p[e0:].reshape(NS, nch1, CHUNK),
    )

    degflat = _deg_call(edges[1], edges[2], edges[4], edges[5], nch0, nch1)
    deg3 = degflat.reshape(NC, N, 1)
    h1p = _mm_scale(deg3, x, W1)
    S1 = _prop_call(h1p, edges, nch0, nch1)
    h2p = _mid_layer(deg3, S1, h1p, b1, W2)
    S2 = _prop_call(h2p, edges, nch0, nch1)
    return _final_layer(deg3, S2, h2p, b2)


# speedup vs baseline: 1.0218x; 1.0007x over previous
import functools

import jax
import jax.numpy as jnp
from jax import lax
from jax.experimental import pallas as pl
from jax.experimental.pallas import tpu as pltpu
from jax.experimental.pallas import tpu_sc as plsc

N = 10000
D = 128
L = 16
NC = 2
NS = 16
NW = NC * NS
CHUNK = 128
ROWS_MAIN = (N // (NS * 8)) * 8
TAIL = N - NS * ROWS_MAIN
TAIL_OFF = NS * ROWS_MAIN
_SLAB = [(o, min(CHUNK, ROWS_MAIN - o)) for o in range(0, ROWS_MAIN, CHUNK)]

_mesh = plsc.VectorSubcoreMesh(core_axis_name="c", subcore_axis_name="s")


def _deg_body(nch0, nch1, dst0_hbm, w0_hbm, dst1_hbm, w1_hbm, deg_out,
              dst_all, w_all, zb, deg_sh):
    c = lax.axis_index("c")
    s = lax.axis_index("s")
    r0 = s * ROWS_MAIN

    def zb_body(i, _):
        zb[pl.ds(i * L, L)] = jnp.zeros((L,), jnp.float32)
        return 0

    lax.fori_loop(0, ROWS_MAIN // L, zb_body, 0)
    pltpu.sync_copy(zb, deg_sh.at[pl.ds(r0, ROWS_MAIN)])

    @pl.when(s == 0)
    def _():
        pltpu.sync_copy(zb.at[pl.ds(0, TAIL)], deg_sh.at[pl.ds(TAIL_OFF, TAIL)])

    plsc.subcore_barrier()

    def run(dst3, w3, nch):
        pltpu.sync_copy(dst3.at[s], dst_all.at[pl.ds(0, nch)])
        pltpu.sync_copy(w3.at[s], w_all.at[pl.ds(0, nch)])

        def chunk_body(k, _):
            pltpu.sync_copy(w_all.at[k], deg_sh.at[dst_all.at[k]], add=True)
            return 0

        lax.fori_loop(0, nch, chunk_body, 0)

    @pl.when(c == 0)
    def _():
        run(dst0_hbm, w0_hbm, nch0)

    @pl.when(c == 1)
    def _():
        run(dst1_hbm, w1_hbm, nch1)

    plsc.subcore_barrier()

    pltpu.sync_copy(deg_sh.at[pl.ds(r0, ROWS_MAIN)], zb)
    pltpu.sync_copy(zb, deg_out.at[pl.ds(c * N + r0, ROWS_MAIN)])

    @pl.when(s == 0)
    def _():
        pltpu.sync_copy(deg_sh.at[pl.ds(TAIL_OFF, TAIL)], zb.at[pl.ds(0, TAIL)])
        pltpu.sync_copy(zb.at[pl.ds(0, TAIL)], deg_out.at[pl.ds(c * N + TAIL_OFF, TAIL)])


def _deg_call(dst0, w0, dst1, w1, nch0, nch1):
    return pl.kernel(
        functools.partial(_deg_body, nch0, nch1),
        out_type=jax.ShapeDtypeStruct((NC * N,), jnp.float32),
        mesh=_mesh,
        scratch_types=[
            pltpu.VMEM((max(nch0, nch1), CHUNK), jnp.int32),
            pltpu.VMEM((max(nch0, nch1), CHUNK), jnp.float32),
            pltpu.VMEM((ROWS_MAIN,), jnp.float32),
            pltpu.VMEM_SHARED((N,), jnp.float32),
        ],
    )(dst0, w0, dst1, w1)


def _prop_body(nch0, nch1, hp_hbm, src0_hbm, dst0_hbm, w0_hbm,
               src1_hbm, dst1_hbm, w1_hbm, part_out,
               dst_all, idx_a, idx_b, w_a, w_b, rows_a, rows_b,
               acc_sh, sem_ga, sem_gb, sem_sa, sem_sb, sem_pa, sem_pb):
    c = lax.axis_index("c")
    s = lax.axis_index("s")
    r0 = s * ROWS_MAIN

    def zero_rows(e, _):
        z = jnp.zeros((L,), jnp.float32)
        for j in range(D // L):
            rows_a[e, pl.ds(j * L, L)] = z
        return 0

    lax.fori_loop(0, CHUNK, zero_rows, 0)
    for o, nr in _SLAB:
        pltpu.sync_copy(rows_a.at[pl.ds(0, nr)], acc_sh.at[pl.ds(r0 + o, nr)])

    @pl.when(s == 0)
    def _():
        pltpu.sync_copy(rows_a.at[pl.ds(0, TAIL)], acc_sh.at[pl.ds(TAIL_OFF, TAIL)])

    plsc.subcore_barrier()

    def gather(idx_r, rows_r, sem):
        pltpu.async_copy(hp_hbm.at[idx_r], rows_r, sem)

    def gather_wait(idx_r, rows_r, sem):
        pltpu.make_async_copy(hp_hbm.at[idx_r], rows_r, sem).wait()

    def scatter(k, rows_r, sem):
        pltpu.async_copy(rows_r, acc_sh.at[dst_all.at[k]], sem, add=True)

    def scatter_wait(k, rows_r, sem):
        pltpu.make_async_copy(rows_r, acc_sh.at[dst_all.at[k]], sem).wait()

    def mul(w_r, rows_r):
        def edge_body(e, _):
            wsc = w_r[pl.ds(e, L)][0]
            for j in range(D // L):
                rows_r[e, pl.ds(j * L, L)] = rows_r[e, pl.ds(j * L, L)] * wsc
            return 0

        lax.fori_loop(0, CHUNK, edge_body, 0, unroll=4)

    def run(src3, dst3, w3, nch):
        last = nch - 1
        pltpu.sync_copy(dst3.at[s], dst_all.at[pl.ds(0, nch)])

        def prefetch(k, idx_r, w_r, sem):
            pltpu.async_copy(src3.at[s, k], idx_r, sem)
            pltpu.async_copy(w3.at[s, k], w_r.at[pl.ds(0, CHUNK)], sem)

        def prefetch_wait(k, idx_r, w_r, sem):
            pltpu.make_async_copy(src3.at[s, k], idx_r, sem).wait()
            pltpu.make_async_copy(w3.at[s, k], w_r.at[pl.ds(0, CHUNK)], sem).wait()

        prefetch(0, idx_a, w_a, sem_pa)
        prefetch_wait(0, idx_a, w_a, sem_pa)
        gather(idx_a, rows_a, sem_ga)
        prefetch(1, idx_b, w_b, sem_pb)

        def pair_body(i, _):
            k = 2 * i

            @pl.when(i > 0)
            def _():
                scatter_wait(k - 1, rows_b, sem_sb)

            prefetch_wait(k + 1, idx_b, w_b, sem_pb)
            gather(idx_b, rows_b, sem_gb)
            gather_wait(idx_a, rows_a, sem_ga)
            mul(w_a, rows_a)
            scatter(k, rows_a, sem_sa)
            prefetch(jnp.minimum(k + 2, last), idx_a, w_a, sem_pa)
            gather_wait(idx_b, rows_b, sem_gb)
            scatter_wait(k, rows_a, sem_sa)
            prefetch_wait(jnp.minimum(k + 2, last), idx_a, w_a, sem_pa)
            gather(idx_a, rows_a, sem_ga)
            mul(w_b, rows_b)
            scatter(k + 1, rows_b, sem_sb)
            prefetch(jnp.minimum(k + 3, last), idx_b, w_b, sem_pb)
            return 0

        lax.fori_loop(0, nch // 2, pair_body, 0)
        scatter_wait(last, rows_b, sem_sb)
        prefetch_wait(last, idx_b, w_b, sem_pb)
        gather_wait(idx_a, rows_a, sem_ga)

    @pl.when(c == 0)
    def _():
        run(src0_hbm, dst0_hbm, w0_hbm, nch0)

    @pl.when(c == 1)
    def _():
        run(src1_hbm, dst1_hbm, w1_hbm, nch1)

    plsc.subcore_barrier()

    for o, nr in _SLAB:
        pltpu.sync_copy(acc_sh.at[pl.ds(r0 + o, nr)], rows_a.at[pl.ds(0, nr)])
        pltpu.sync_copy(rows_a.at[pl.ds(0, nr)], part_out.at[c, pl.ds(r0 + o, nr)])

    @pl.when(s == 0)
    def _():
        pltpu.sync_copy(acc_sh.at[pl.ds(TAIL_OFF, TAIL)], rows_a.at[pl.ds(0, TAIL)])
        pltpu.sync_copy(rows_a.at[pl.ds(0, TAIL)], part_out.at[c, pl.ds(TAIL_OFF, TAIL)])


def _prop_call(hp, edges, nch0, nch1):
    src0, dst0, w0, src1, dst1, w1 = edges
    return pl.kernel(
        functools.partial(_prop_body, nch0, nch1),
        out_type=jax.ShapeDtypeStruct((NC, N, D), jnp.float32),
        mesh=_mesh,
        scratch_types=[
            pltpu.VMEM((max(nch0, nch1), CHUNK), jnp.int32),
            pltpu.VMEM((CHUNK,), jnp.int32),
            pltpu.VMEM((CHUNK,), jnp.int32),
            pltpu.VMEM((CHUNK + L,), jnp.float32),
            pltpu.VMEM((CHUNK + L,), jnp.float32),
            pltpu.VMEM((CHUNK, D), jnp.float32),
            pltpu.VMEM((CHUNK, D), jnp.float32),
            pltpu.VMEM_SHARED((N, D), jnp.float32),
            pltpu.SemaphoreType.DMA,
            pltpu.SemaphoreType.DMA,
            pltpu.SemaphoreType.DMA,
            pltpu.SemaphoreType.DMA,
            pltpu.SemaphoreType.DMA,
            pltpu.SemaphoreType.DMA,
        ],
    )(hp, src0, dst0, w0, src1, dst1, w1)


_BLK = 1000


def _dinv_block(deg_ref):
    deg = deg_ref[0, :, 0] + deg_ref[1, :, 0] + 1.0
    return jnp.where(deg > 0, lax.rsqrt(deg), 0.0)


def _mm_scale_body(deg_ref, x_ref, w_ref, o_ref):
    dinv = _dinv_block(deg_ref)
    h = jnp.dot(x_ref[...], w_ref[...], preferred_element_type=jnp.float32)
    o_ref[...] = h * dinv[:, None]


def _mm_scale(deg3, x, W1):
    return pl.pallas_call(
        _mm_scale_body,
        grid=(N // _BLK,),
        in_specs=[
            pl.BlockSpec((NC, _BLK, 1), lambda i: (0, i, 0)),
            pl.BlockSpec((_BLK, D), lambda i: (i, 0)),
            pl.BlockSpec((D, D), lambda i: (0, 0)),
        ],
        out_specs=pl.BlockSpec((_BLK, D), lambda i: (i, 0)),
        out_shape=jax.ShapeDtypeStruct((N, D), jnp.float32),
    )(deg3, x, W1)


def _mid_body(deg_ref, s_ref, hp_ref, b_ref, w_ref, o_ref):
    dinv = _dinv_block(deg_ref)
    pre = (s_ref[0] + s_ref[1] + hp_ref[...]) * dinv[:, None] + b_ref[...]
    h1 = jnp.maximum(pre, 0.0)
    g = jnp.dot(h1, w_ref[...], preferred_element_type=jnp.float32)
    o_ref[...] = g * dinv[:, None]


def _mid_layer(deg3, S1, h1p, b1, W2):
    return pl.pallas_call(
        _mid_body,
        grid=(N // _BLK,),
        in_specs=[
            pl.BlockSpec((NC, _BLK, 1), lambda i: (0, i, 0)),
            pl.BlockSpec((NC, _BLK, D), lambda i: (0, i, 0)),
            pl.BlockSpec((_BLK, D), lambda i: (i, 0)),
            pl.BlockSpec((1, D), lambda i: (0, 0)),
            pl.BlockSpec((D, D), lambda i: (0, 0)),
        ],
        out_specs=pl.BlockSpec((_BLK, D), lambda i: (i, 0)),
        out_shape=jax.ShapeDtypeStruct((N, D), jnp.float32),
    )(deg3, S1, h1p, b1.reshape(1, D), W2)


def _final_body(deg_ref, s_ref, hp_ref, b_ref, o_ref):
    dinv = _dinv_block(deg_ref)
    o_ref[...] = (s_ref[0] + s_ref[1] + hp_ref[...]) * dinv[:, None] + b_ref[...]


def _final_layer(deg3, S2, h2p, b2):
    return pl.pallas_call(
        _final_body,
        grid=(N // _BLK,),
        in_specs=[
            pl.BlockSpec((NC, _BLK, 1), lambda i: (0, i, 0)),
            pl.BlockSpec((NC, _BLK, D), lambda i: (0, i, 0)),
            pl.BlockSpec((_BLK, D), lambda i: (i, 0)),
            pl.BlockSpec((1, D), lambda i: (0, 0)),
        ],
        out_specs=pl.BlockSpec((_BLK, D), lambda i: (i, 0)),
        out_shape=jax.ShapeDtypeStruct((N, D), jnp.float32),
    )(deg3, S2, h2p, b2.reshape(1, D))


_CORE0_FRAC = 0.76


def kernel(x, attn_edge_index, attn_edge_weight, W1, b1, W2, b2):
    E = attn_edge_index.shape[1]
    t = -(-E // (NS * CHUNK))
    nch0 = max(2, int(round(t * _CORE0_FRAC / 2)) * 2)
    nch1 = max(2, ((t - nch0 + 1) // 2) * 2)
    e0 = NS * nch0 * CHUNK
    e_pad = e0 + NS * nch1 * CHUNK
    pad = e_pad - E

    src_p = jnp.pad(attn_edge_index[0], (0, pad))
    dst_p = jnp.pad(attn_edge_index[1], (0, pad))
    w_p = jnp.pad(attn_edge_weight, (0, pad))
    edges = (
        src_p[:e0].reshape(NS, nch0, CHUNK),
        dst_p[:e0].reshape(NS, nch0, CHUNK),
        w_p[:e0].reshape(NS, nch0, CHUNK),
        src_p[e0:].reshape(NS, nch1, CHUNK),
        dst_p[e0:].reshape(NS, nch1, CHUNK),
        w_p[e0:].reshape(NS, nch1, CHUNK),
    )

    degflat = _deg_call(edges[1], edges[2], edges[4], edges[5], nch0, nch1)
    deg3 = degflat.reshape(NC, N, 1)
    h1p = _mm_scale(deg3, x, W1)
    S1 = _prop_call(h1p, edges, nch0, nch1)
    h2p = _mid_layer(deg3, S1, h1p, b1, W2)
    S2 = _prop_call(h2p, edges, nch0, nch1)
    return _final_layer(deg3, S2, h2p, b2)
